# baseline (speedup 1.0000x reference)
import functools

import jax
import jax.numpy as jnp
from jax import lax
from jax.experimental import pallas as pl
from jax.experimental.pallas import tpu as pltpu
from jax.experimental.pallas import tpu_sc as plsc

B_G = 128
NPG = 64
N_EDGES = 65536
D = 64
DP = 2 * D
P = B_G * NPG * NPG
PR = P // 2
NC = 2
NS = 16
L = 16
EPT = N_EDGES // NS
CHUNK_G = 4
CHUNK_PROWS = CHUNK_G * NPG * NPG // 2
N_CHUNKS = (B_G // NC) // CHUNK_G
DUMP = CHUNK_PROWS
GB = 128
CAP = 768
PROWS_PER_TILE = CHUNK_PROWS // NS
ZR = 128
FPW = 2 * P // (NC * NS)


def _body(src_hbm, dst_hbm, ea_hbm, ib_hbm, ei_out, attr_out,
          src_v, dst_v, bucket, ovf, cnts, ocnt, ebuf, rbuf, e16, r16,
          gbuf, g16, zbuf, bbuf, bidx, eibuf, chunk, sem):
    c = lax.axis_index("c")
    s = lax.axis_index("s")
    w = s * NC + c
    iot = lax.iota(jnp.int32, L)

    pltpu.sync_copy(src_hbm.at[pl.ds(pl.multiple_of(s * EPT, EPT), EPT)], src_v)
    pltpu.sync_copy(dst_hbm.at[pl.ds(pl.multiple_of(s * EPT, EPT), EPT)], dst_v)

    def _zb(i, carry):
        z = jnp.zeros((L,), jnp.float32)
        for q in range(DP // L):
            zbuf[i, pl.ds(q * L, L)] = z
        return carry
    lax.fori_loop(0, ZR, _zb, 0, unroll=4)

    bidx[0] = iot & 1
    pltpu.async_copy(ib_hbm.at[bidx.at[0]], bbuf, sem).wait()

    cnts[pl.ds(0, L)] = jnp.zeros((L,), jnp.int32)
    ocnt[pl.ds(0, L)] = jnp.zeros((L,), jnp.int32)
    g_base = c * (B_G // NC)
    zv = jnp.zeros((L,), jnp.int32)

    def _bucket(i, carry):
        sv = src_v[pl.ds(i * L, L)]
        gl = (sv >> 6) - g_base
        valid = (gl >= 0) & (gl < (B_G // NC))
        k_e = (gl >> 2) & (N_CHUNKS - 1)
        rank, _ = plsc.scan_count(k_e, mask=valid)
        base = plsc.load_gather(cnts, [k_e], mask=valid)
        pos = base + rank - 1
        fits = valid & (pos < CAP)
        eid = i * L + iot
        plsc.store_scatter(bucket, [k_e, pos], eid, mask=fits)
        plsc.store_scatter(cnts, [k_e], jnp.minimum(pos + 1, CAP),
                           mask=valid)
        over = valid & (pos >= CAP)
        orank, _ = plsc.scan_count(zv, mask=over)
        obase = plsc.load_gather(ocnt, [zv], mask=over)
        opos = obase + orank - 1
        plsc.store_scatter(ovf, [opos], eid, mask=over)
        plsc.store_scatter(ocnt, [zv], opos + 1, mask=over)
        return carry
    lax.fori_loop(0, EPT // L, _bucket, 0, unroll=2)

    base_f = w * FPW

    def _ei_batch(b, carry):
        def _row(i, cc):
            f = base_f + b * 2048 + i * L + iot
            q = f - P
            val = jnp.where(f < P, f >> 6,
                            ((q >> 12) << 6) | (q & (NPG - 1)))
            eibuf[i >> 3, pl.ds((i & 7) * L, L)] = val
            return cc
        lax.fori_loop(0, 2048 // L, _row, 0, unroll=8)
        pltpu.sync_copy(eibuf,
                        ei_out.at[pl.ds(pl.multiple_of(
                            (base_f + b * 2048) >> 7, 16), 16)])
        return carry
    lax.fori_loop(0, FPW // 2048, _ei_batch, 0)

    for m in range(PROWS_PER_TILE // ZR):
        pltpu.sync_copy(zbuf,
                        chunk.at[pl.ds(pl.multiple_of(
                            s * PROWS_PER_TILE + m * ZR, ZR), ZR)])
    plsc.subcore_barrier()

    cv = cnts[pl.ds(0, L)]
    nov = ocnt[pl.ds(0, L)][0]
    novb = (nov + L - 1) >> 4

    for k in range(N_CHUNKS):
        n_k = cv[k]
        nb = (n_k + GB - 1) >> 7

        def _gs(j, cc, k=k, n_k=n_k):
            be = j * GB
            for q in range(GB // L):
                e = bucket[k, pl.ds(be + q * L, L)]
                m = (be + q * L + iot) < n_k
                e = jnp.where(m, e, 0)
                sv = plsc.load_gather(src_v, [e])
                dv = plsc.load_gather(dst_v, [e])
                gl = (sv >> 6) - g_base
                row = ((gl & (CHUNK_G - 1)) << 12) \
                    | ((sv & (NPG - 1)) << 6) | (dv & (NPG - 1))
                ebuf[0, pl.ds(q * L, L)] = e + s * EPT
                rbuf[0, pl.ds(q * L, L)] = jnp.where(m, row >> 1, DUMP)
            pltpu.async_copy(ea_hbm.at[ebuf.at[0]], gbuf, sem).wait()
            pltpu.sync_copy(gbuf, chunk.at[rbuf.at[0]], add=True)
            return cc
        lax.fori_loop(0, nb, _gs, 0)

        def _ov(j, cc, k=k):
            e = ovf[pl.ds(j * L, L)]
            mv = (j * L + iot) < nov
            e = jnp.where(mv, e, 0)
            sv = plsc.load_gather(src_v, [e])
            dv = plsc.load_gather(dst_v, [e])
            gl = (sv >> 6) - g_base
            k_e = (gl >> 2) & (N_CHUNKS - 1)
            mm = mv & (k_e == k) & (gl >= 0) & (gl < (B_G // NC))
            row = ((gl & (CHUNK_G - 1)) << 12) \
                | ((sv & (NPG - 1)) << 6) | (dv & (NPG - 1))
            e16[0] = jnp.where(mm, e + s * EPT, 0)
            r16[0] = jnp.where(mm, row >> 1, DUMP)
            pltpu.async_copy(ea_hbm.at[e16.at[0]], g16, sem).wait()
            pltpu.sync_copy(g16, chunk.at[r16.at[0]], add=True)
            return cc
        lax.fori_loop(0, novb, _ov, 0)

        dd = s * L + iot
        gi = dd >> 6
        a = dd & (NPG - 1)
        bidx[0] = ((gi << 12) | (a * (NPG + 1))) >> 1
        pltpu.sync_copy(bbuf, chunk.at[bidx.at[0]], add=True)

        plsc.subcore_barrier()

        chunk_base = (g_base + k * CHUNK_G) * (NPG * NPG // 2)
        pltpu.sync_copy(chunk.at[pl.ds(pl.multiple_of(
                            s * PROWS_PER_TILE, PROWS_PER_TILE),
                            PROWS_PER_TILE)],
                        attr_out.at[pl.ds(pl.multiple_of(
                            chunk_base + s * PROWS_PER_TILE,
                            PROWS_PER_TILE), PROWS_PER_TILE)])
        for m in range(PROWS_PER_TILE // ZR):
            pltpu.sync_copy(zbuf,
                            chunk.at[pl.ds(pl.multiple_of(
                                s * PROWS_PER_TILE + m * ZR, ZR), ZR)])
        plsc.subcore_barrier()


_sc_call = functools.partial(
    pl.kernel,
    mesh=plsc.VectorSubcoreMesh(core_axis_name="c", subcore_axis_name="s"),
    compiler_params=pltpu.CompilerParams(needs_layout_passes=False,
                                         use_tc_tiling_on_sc=True),
    out_type=[jax.ShapeDtypeStruct((2 * P // DP, DP), jnp.int32),
              jax.ShapeDtypeStruct((PR, DP), jnp.float32)],
    scratch_types=[
        pltpu.VMEM((EPT,), jnp.int32),
        pltpu.VMEM((EPT,), jnp.int32),
        pltpu.VMEM((N_CHUNKS, CAP), jnp.int32),
        pltpu.VMEM((EPT,), jnp.int32),
        pltpu.VMEM((L,), jnp.int32),
        pltpu.VMEM((L,), jnp.int32),
        pltpu.VMEM((1, GB), jnp.int32),
        pltpu.VMEM((1, GB), jnp.int32),
        pltpu.VMEM((1, L), jnp.int32),
        pltpu.VMEM((1, L), jnp.int32),
        pltpu.VMEM((GB, DP), jnp.float32),
        pltpu.VMEM((L, DP), jnp.float32),
        pltpu.VMEM((ZR, DP), jnp.float32),
        pltpu.VMEM((L, DP), jnp.float32),
        pltpu.VMEM((1, L), jnp.int32),
        pltpu.VMEM((16, DP), jnp.int32),
        pltpu.VMEM_SHARED((CHUNK_PROWS + 8, DP), jnp.float32),
        pltpu.SemaphoreType.DMA,
    ],
)(_body)


def kernel(edge_index, edge_attr, batch, num_nodes, identity_bias):
    src = edge_index[0]
    dst = edge_index[1]
    odd = (dst & 1)[:, None] == 1
    ea_pair = jnp.concatenate(
        [jnp.where(odd, 0.0, edge_attr), jnp.where(odd, edge_attr, 0.0)],
        axis=1)
    zb = jnp.zeros_like(identity_bias)
    ib_pair = jnp.concatenate(
        [jnp.concatenate([identity_bias, zb], axis=1),
         jnp.concatenate([zb, identity_bias], axis=1),
         jnp.zeros((6, DP), jnp.float32)], axis=0)
    ei2, attr2 = _sc_call(src, dst, ea_pair, ib_pair)
    return ei2.reshape(2, P), attr2.reshape(P, D)


# speedup vs baseline: 1.2059x; 1.2059x over previous
import functools

import jax
import jax.numpy as jnp
from jax import lax
from jax.experimental import pallas as pl
from jax.experimental.pallas import tpu as pltpu
from jax.experimental.pallas import tpu_sc as plsc

B_G = 128
NPG = 64
N_EDGES = 65536
D = 64
P = B_G * NPG * NPG
NC = 2
NS = 16
L = 16
EPT = N_EDGES // NS
CHUNK_G = 4
CHUNK_ROWS = CHUNK_G * NPG * NPG
N_CHUNKS = (B_G // NC) // CHUNK_G
DUMP = CHUNK_ROWS
GB = 128
CAP = 896
ROWS_PER_TILE = CHUNK_ROWS // NS
ZR = 256
PPW = P // (NC * NS)


def _body(src_hbm, dst_hbm, ea_hbm, ib_hbm, ei_out, attr_out,
          src_v, dst_v, bucket, ovf, cnts, ocnt, ebuf, rbuf, e16, r16,
          gbuf, g16, zbuf, bbuf, bidx, eibuf, chunk, sem, semg):
    c = lax.axis_index("c")
    s = lax.axis_index("s")
    w = s * NC + c
    iot = lax.iota(jnp.int32, L)

    pltpu.sync_copy(src_hbm.at[pl.ds(s * EPT, EPT)], src_v)
    pltpu.sync_copy(dst_hbm.at[pl.ds(s * EPT, EPT)], dst_v)

    def _zb(i, carry):
        z = jnp.zeros((L,), jnp.float32)
        for q in range(D // L):
            zbuf[i, pl.ds(q * L, L)] = z
        return carry
    lax.fori_loop(0, ZR, _zb, 0, unroll=4)

    bidx[0] = jnp.zeros((L,), jnp.int32)
    pltpu.async_copy(ib_hbm.at[bidx.at[0]], bbuf, sem).wait()

    cnts[pl.ds(0, L)] = jnp.zeros((L,), jnp.int32)
    ocnt[pl.ds(0, L)] = jnp.zeros((L,), jnp.int32)
    g_base = c * (B_G // NC)
    zv = jnp.zeros((L,), jnp.int32)

    def _bucket(i, carry):
        sv = src_v[pl.ds(i * L, L)]
        gl = (sv >> 6) - g_base
        valid = (gl >= 0) & (gl < (B_G // NC))
        k_e = (gl >> 2) & (N_CHUNKS - 1)
        rank, _ = plsc.scan_count(k_e, mask=valid)
        base = plsc.load_gather(cnts, [k_e], mask=valid)
        pos = base + rank - 1
        fits = valid & (pos < CAP)
        eid = i * L + iot
        plsc.store_scatter(bucket, [k_e, pos], eid, mask=fits)
        plsc.store_scatter(cnts, [k_e], jnp.minimum(pos + 1, CAP),
                           mask=valid)
        over = valid & (pos >= CAP)
        orank, _ = plsc.scan_count(zv, mask=over)
        obase = plsc.load_gather(ocnt, [zv], mask=over)
        opos = obase + orank - 1
        plsc.store_scatter(ovf, [opos], eid, mask=over)
        plsc.store_scatter(ocnt, [zv], opos + 1, mask=over)
        return carry
    lax.fori_loop(0, EPT // L, _bucket, 0, unroll=2)

    base_p = w * PPW

    def _ei_batch(b, carry):
        def _row0(i, cc):
            p = base_p + b * 2048 + i * L + iot
            eibuf[pl.ds(i * L, L)] = p >> 6
            return cc
        lax.fori_loop(0, 2048 // L, _row0, 0, unroll=8)
        pltpu.sync_copy(eibuf, ei_out.at[0, pl.ds(base_p + b * 2048, 2048)])

        def _row1(i, cc):
            p = base_p + b * 2048 + i * L + iot
            eibuf[pl.ds(i * L, L)] = ((p >> 12) << 6) | (p & (NPG - 1))
            return cc
        lax.fori_loop(0, 2048 // L, _row1, 0, unroll=8)
        pltpu.sync_copy(eibuf, ei_out.at[1, pl.ds(base_p + b * 2048, 2048)])
        return carry
    lax.fori_loop(0, PPW // 2048, _ei_batch, 0)

    for m in range(ROWS_PER_TILE // ZR):
        pltpu.sync_copy(zbuf, chunk.at[pl.ds(s * ROWS_PER_TILE + m * ZR, ZR)])
    plsc.subcore_barrier()

    cv = cnts[pl.ds(0, L)]
    nov = ocnt[pl.ds(0, L)][0]
    novb = (nov + L - 1) >> 4

    for k in range(N_CHUNKS):
        n_k = cv[k]
        nb = (n_k + GB - 1) >> 7

        def _build(j, k=k, n_k=n_k):
            pp = j & 1
            be = j * GB
            for q in range(GB // L):
                e = bucket[k, pl.ds(be + q * L, L)]
                m = (be + q * L + iot) < n_k
                e = jnp.where(m, e, 0)
                sv = plsc.load_gather(src_v, [e])
                dv = plsc.load_gather(dst_v, [e])
                gl = (sv >> 6) - g_base
                row = ((gl & (CHUNK_G - 1)) << 12) \
                    | ((sv & (NPG - 1)) << 6) | (dv & (NPG - 1))
                ebuf[pp, pl.ds(q * L, L)] = e + s * EPT
                rbuf[pp, pl.ds(q * L, L)] = jnp.where(m, row, DUMP)

        dd = s * L + iot
        gi = dd >> 6
        a = dd & (NPG - 1)
        bidx[0] = (gi << 12) | (a * (NPG + 1))
        pltpu.sync_copy(bbuf, chunk.at[bidx.at[0]], add=True)

        @pl.when(nb > 0)
        def _():
            _build(0)
            pltpu.async_copy(ea_hbm.at[ebuf.at[0]], gbuf.at[0], semg.at[0])

        def _gs(j, cc, k=k, n_k=n_k):
            p = j & 1

            @pl.when(j + 1 < nb)
            def _():
                _build(j + 1)
                pltpu.async_copy(ea_hbm.at[ebuf.at[1 - p]], gbuf.at[1 - p],
                                 semg.at[1 - p])
            pltpu.make_async_copy(ea_hbm.at[ebuf.at[p]], gbuf.at[p],
                                  semg.at[p]).wait()
            pltpu.sync_copy(gbuf.at[p], chunk.at[rbuf.at[p]], add=True)
            return cc
        lax.fori_loop(0, nb, _gs, 0)

        def _ov(j, cc, k=k):
            e = ovf[pl.ds(j * L, L)]
            mv = (j * L + iot) < nov
            e = jnp.where(mv, e, 0)
            sv = plsc.load_gather(src_v, [e])
            dv = plsc.load_gather(dst_v, [e])
            gl = (sv >> 6) - g_base
            k_e = (gl >> 2) & (N_CHUNKS - 1)
            mm = mv & (k_e == k) & (gl >= 0) & (gl < (B_G // NC))
            row = ((gl & (CHUNK_G - 1)) << 12) \
                | ((sv & (NPG - 1)) << 6) | (dv & (NPG - 1))
            e16[0] = jnp.where(mm, e + s * EPT, 0)
            r16[0] = jnp.where(mm, row, DUMP)
            pltpu.async_copy(ea_hbm.at[e16.at[0]], g16, sem).wait()
            pltpu.sync_copy(g16, chunk.at[r16.at[0]], add=True)
            return cc
        lax.fori_loop(0, novb, _ov, 0)

        plsc.subcore_barrier()

        chunk_base = (g_base + k * CHUNK_G) * (NPG * NPG)
        pltpu.sync_copy(chunk.at[pl.ds(s * ROWS_PER_TILE, ROWS_PER_TILE)],
                        attr_out.at[pl.ds(chunk_base + s * ROWS_PER_TILE,
                                          ROWS_PER_TILE)])
        for m in range(ROWS_PER_TILE // ZR):
            pltpu.sync_copy(zbuf,
                            chunk.at[pl.ds(s * ROWS_PER_TILE + m * ZR, ZR)])
        plsc.subcore_barrier()


_sc_call = functools.partial(
    pl.kernel,
    mesh=plsc.VectorSubcoreMesh(core_axis_name="c", subcore_axis_name="s"),
    compiler_params=pltpu.CompilerParams(needs_layout_passes=False,
                                         use_tc_tiling_on_sc=False),
    out_type=[jax.ShapeDtypeStruct((2, P), jnp.int32),
              jax.ShapeDtypeStruct((P, D), jnp.float32)],
    scratch_types=[
        pltpu.VMEM((EPT,), jnp.int32),
        pltpu.VMEM((EPT,), jnp.int32),
        pltpu.VMEM((N_CHUNKS, CAP), jnp.int32),
        pltpu.VMEM((EPT,), jnp.int32),
        pltpu.VMEM((L,), jnp.int32),
        pltpu.VMEM((L,), jnp.int32),
        pltpu.VMEM((2, GB), jnp.int32),
        pltpu.VMEM((2, GB), jnp.int32),
        pltpu.VMEM((1, L), jnp.int32),
        pltpu.VMEM((1, L), jnp.int32),
        pltpu.VMEM((2, GB, D), jnp.float32),
        pltpu.VMEM((L, D), jnp.float32),
        pltpu.VMEM((ZR, D), jnp.float32),
        pltpu.VMEM((L, D), jnp.float32),
        pltpu.VMEM((1, L), jnp.int32),
        pltpu.VMEM((2048,), jnp.int32),
        pltpu.VMEM_SHARED((CHUNK_ROWS + 8, D), jnp.float32),
        pltpu.SemaphoreType.DMA,
        pltpu.SemaphoreType.DMA((2,)),
    ],
)(_body)


def kernel(edge_index, edge_attr, batch, num_nodes, identity_bias):
    src = edge_index[0]
    dst = edge_index[1]
    out_ei, out_attr = _sc_call(src, dst, edge_attr, identity_bias)
    return out_ei, out_attr


# speedup vs baseline: 1.2154x; 1.0079x over previous
import functools

import jax
import jax.numpy as jnp
from jax import lax
from jax.experimental import pallas as pl
from jax.experimental.pallas import tpu as pltpu
from jax.experimental.pallas import tpu_sc as plsc

B_G = 128
NPG = 64
N_EDGES = 65536
D = 64
P = B_G * NPG * NPG
NC = 2
NS = 16
L = 16
EPT = N_EDGES // NS
CHUNK_G = 4
CHUNK_ROWS = CHUNK_G * NPG * NPG
N_CHUNKS = (B_G // NC) // CHUNK_G
DUMP = CHUNK_ROWS
GB = 128
CAP = 896
ROWS_PER_TILE = CHUNK_ROWS // NS
ZR = 256
PPW = P // (NC * NS)


def _body(src_hbm, dst_hbm, ea_hbm, ib_hbm, attr_out,
          src_v, dst_v, bucket, ovf, cnts, ocnt, ebuf, rbuf, e16, r16,
          gbuf, g16, zbuf, bbuf, bidx, chunk, sem, semg):
    c = lax.axis_index("c")
    s = lax.axis_index("s")
    w = s * NC + c
    iot = lax.iota(jnp.int32, L)

    pltpu.sync_copy(src_hbm.at[pl.ds(s * EPT, EPT)], src_v)
    pltpu.sync_copy(dst_hbm.at[pl.ds(s * EPT, EPT)], dst_v)

    def _zb(i, carry):
        z = jnp.zeros((L,), jnp.float32)
        for q in range(D // L):
            zbuf[i, pl.ds(q * L, L)] = z
        return carry
    lax.fori_loop(0, ZR, _zb, 0, unroll=4)

    bidx[0] = jnp.zeros((L,), jnp.int32)
    pltpu.async_copy(ib_hbm.at[bidx.at[0]], bbuf, sem).wait()

    cnts[pl.ds(0, L)] = jnp.zeros((L,), jnp.int32)
    ocnt[pl.ds(0, L)] = jnp.zeros((L,), jnp.int32)
    g_base = c * (B_G // NC)
    zv = jnp.zeros((L,), jnp.int32)

    def _bucket(i, carry):
        sv = src_v[pl.ds(i * L, L)]
        gl = (sv >> 6) - g_base
        valid = (gl >= 0) & (gl < (B_G // NC))
        k_e = (gl >> 2) & (N_CHUNKS - 1)
        rank, _ = plsc.scan_count(k_e, mask=valid)
        base = plsc.load_gather(cnts, [k_e], mask=valid)
        pos = base + rank - 1
        fits = valid & (pos < CAP)
        eid = i * L + iot
        plsc.store_scatter(bucket, [k_e, pos], eid, mask=fits)
        plsc.store_scatter(cnts, [k_e], jnp.minimum(pos + 1, CAP),
                           mask=valid)
        over = valid & (pos >= CAP)
        orank, _ = plsc.scan_count(zv, mask=over)
        obase = plsc.load_gather(ocnt, [zv], mask=over)
        opos = obase + orank - 1
        plsc.store_scatter(ovf, [opos], eid, mask=over)
        plsc.store_scatter(ocnt, [zv], opos + 1, mask=over)
        return carry
    lax.fori_loop(0, EPT // L, _bucket, 0, unroll=2)

    for m in range(ROWS_PER_TILE // ZR):
        pltpu.sync_copy(zbuf, chunk.at[pl.ds(s * ROWS_PER_TILE + m * ZR, ZR)])
    plsc.subcore_barrier()

    cv = cnts[pl.ds(0, L)]
    nov = ocnt[pl.ds(0, L)][0]
    novb = (nov + L - 1) >> 4

    for k in range(N_CHUNKS):
        n_k = cv[k]
        nb = (n_k + GB - 1) >> 7

        def _build(j, k=k, n_k=n_k):
            pp = j & 1
            be = j * GB
            for q in range(GB // L):
                e = bucket[k, pl.ds(be + q * L, L)]
                m = (be + q * L + iot) < n_k
                e = jnp.where(m, e, 0)
                sv = plsc.load_gather(src_v, [e])
                dv = plsc.load_gather(dst_v, [e])
                gl = (sv >> 6) - g_base
                row = ((gl & (CHUNK_G - 1)) << 12) \
                    | ((sv & (NPG - 1)) << 6) | (dv & (NPG - 1))
                ebuf[pp, pl.ds(q * L, L)] = e + s * EPT
                rbuf[pp, pl.ds(q * L, L)] = jnp.where(m, row, DUMP)

        dd = s * L + iot
        gi = dd >> 6
        a = dd & (NPG - 1)
        bidx[0] = (gi << 12) | (a * (NPG + 1))
        pltpu.sync_copy(bbuf, chunk.at[bidx.at[0]], add=True)

        @pl.when(nb > 0)
        def _():
            _build(0)
            pltpu.async_copy(ea_hbm.at[ebuf.at[0]], gbuf.at[0], semg.at[0])

        def _gs(j, cc, k=k, n_k=n_k):
            p = j & 1

            @pl.when(j + 1 < nb)
            def _():
                _build(j + 1)
                pltpu.async_copy(ea_hbm.at[ebuf.at[1 - p]], gbuf.at[1 - p],
                                 semg.at[1 - p])
            pltpu.make_async_copy(ea_hbm.at[ebuf.at[p]], gbuf.at[p],
                                  semg.at[p]).wait()
            pltpu.sync_copy(gbuf.at[p], chunk.at[rbuf.at[p]], add=True)
            return cc
        lax.fori_loop(0, nb, _gs, 0)

        def _ov(j, cc, k=k):
            e = ovf[pl.ds(j * L, L)]
            mv = (j * L + iot) < nov
            e = jnp.where(mv, e, 0)
            sv = plsc.load_gather(src_v, [e])
            dv = plsc.load_gather(dst_v, [e])
            gl = (sv >> 6) - g_base
            k_e = (gl >> 2) & (N_CHUNKS - 1)
            mm = mv & (k_e == k) & (gl >= 0) & (gl < (B_G // NC))
            row = ((gl & (CHUNK_G - 1)) << 12) \
                | ((sv & (NPG - 1)) << 6) | (dv & (NPG - 1))
            e16[0] = jnp.where(mm, e + s * EPT, 0)
            r16[0] = jnp.where(mm, row, DUMP)
            pltpu.async_copy(ea_hbm.at[e16.at[0]], g16, sem).wait()
            pltpu.sync_copy(g16, chunk.at[r16.at[0]], add=True)
            return cc
        lax.fori_loop(0, novb, _ov, 0)

        plsc.subcore_barrier()

        chunk_base = (g_base + k * CHUNK_G) * (NPG * NPG)
        pltpu.sync_copy(chunk.at[pl.ds(s * ROWS_PER_TILE, ROWS_PER_TILE)],
                        attr_out.at[pl.ds(chunk_base + s * ROWS_PER_TILE,
                                          ROWS_PER_TILE)])
        for m in range(ROWS_PER_TILE // ZR):
            pltpu.sync_copy(zbuf,
                            chunk.at[pl.ds(s * ROWS_PER_TILE + m * ZR, ZR)])
        plsc.subcore_barrier()


_sc_call = functools.partial(
    pl.kernel,
    mesh=plsc.VectorSubcoreMesh(core_axis_name="c", subcore_axis_name="s"),
    compiler_params=pltpu.CompilerParams(needs_layout_passes=False,
                                         use_tc_tiling_on_sc=False),
    out_type=[jax.ShapeDtypeStruct((P, D), jnp.float32)],
    scratch_types=[
        pltpu.VMEM((EPT,), jnp.int32),
        pltpu.VMEM((EPT,), jnp.int32),
        pltpu.VMEM((N_CHUNKS, CAP), jnp.int32),
        pltpu.VMEM((EPT,), jnp.int32),
        pltpu.VMEM((L,), jnp.int32),
        pltpu.VMEM((L,), jnp.int32),
        pltpu.VMEM((2, GB), jnp.int32),
        pltpu.VMEM((2, GB), jnp.int32),
        pltpu.VMEM((1, L), jnp.int32),
        pltpu.VMEM((1, L), jnp.int32),
        pltpu.VMEM((2, GB, D), jnp.float32),
        pltpu.VMEM((L, D), jnp.float32),
        pltpu.VMEM((ZR, D), jnp.float32),
        pltpu.VMEM((L, D), jnp.float32),
        pltpu.VMEM((1, L), jnp.int32),
        pltpu.VMEM_SHARED((CHUNK_ROWS + 8, D), jnp.float32),
        pltpu.SemaphoreType.DMA,
        pltpu.SemaphoreType.DMA((2,)),
    ],
)(_body)


def kernel(edge_index, edge_attr, batch, num_nodes, identity_bias):
    src = edge_index[0]
    dst = edge_index[1]
    (out_attr,) = _sc_call(src, dst, edge_attr, identity_bias)
    pp = jnp.arange(P, dtype=jnp.int32)
    out_ei = jnp.stack([pp >> 6, ((pp >> 12) << 6) | (pp & (NPG - 1))])
    return out_ei, out_attr
